# baseline (speedup 1.0000x reference)
import functools

import jax
import jax.numpy as jnp
from jax import lax
from jax.experimental import pallas as pl
from jax.experimental.pallas import tpu as pltpu
from jax.experimental.pallas import tpu_sc as plsc

B, S = 4, 8192
D = 64
A = 8
V = 12
VPAD = 16
ROW = A * D
N = B * S
TPR = 16
LWROWS = N // TPR

_info = plsc.get_sparse_core_info()
_NC, _NS = _info.num_cores, _info.num_subcores
NW = _NC * _NS
TPW = N // NW
CH = 64
NCH = TPW // CH


def _prep_body(dig_ref, eos_ref, empty_ref, off_ref, tok_ref, lwt_ref,
               tokw_ref, tbl_ref, lw_ref, tokoff_ref):
    rid = lax.broadcasted_iota(jnp.int32, (VPAD, D), 0)
    base = jnp.where(rid == 1, eos_ref[...], jnp.zeros((VPAD, D), jnp.float32))
    base = jnp.where(rid == 2, empty_ref[...], base)
    for k in range(V - 3):
        base = jnp.where(rid == 3 + k, dig_ref[k:k + 1, :], base)
    tbl = base[:, None, :] + off_ref[...][None, :, :]
    tbl_ref[...] = jnp.concatenate([tbl] * NW, axis=0)

    tokoff_ref[...] = (tokw_ref[...]
                       + lax.broadcasted_iota(jnp.int32, (NW, TPW), 0) * VPAD)

    toks = tok_ref[...].astype(jnp.float32)
    rep = (lax.broadcasted_iota(jnp.int32, (TPR, TPR * A), 1) // A
           == lax.broadcasted_iota(jnp.int32, (TPR, TPR * A), 0)
           ).astype(jnp.float32)
    trep = jax.lax.dot_general(toks, rep, (((1,), (0,)), ((), ())),
                               preferred_element_type=jnp.float32)
    lwt_tiled = jnp.concatenate([lwt_ref[...]] * TPR, axis=1)
    acc = jnp.where(trep == 0.0, lwt_tiled[0:1, :],
                    jnp.zeros((LWROWS, TPR * A), jnp.float32))
    for t in range(1, V):
        acc = jnp.where(trep == float(t), lwt_tiled[t:t + 1, :], acc)
    lw_ref[...] = acc


_prep = pl.pallas_call(
    _prep_body,
    out_shape=[
        jax.ShapeDtypeStruct((NW * VPAD, A, D), jnp.float32),
        jax.ShapeDtypeStruct((LWROWS, TPR * A), jnp.float32),
        jax.ShapeDtypeStruct((NW, TPW), jnp.int32),
    ],
)


def _sc_body(tbl_hbm, tok_hbm, atoms_hbm, idx_v, rows_v, sem_g0, sem_g1,
             sem_s0, sem_s1):
    wid = lax.axis_index("s") * _NC + lax.axis_index("c")
    row_base = wid * TPW
    pltpu.sync_copy(tok_hbm.at[wid], idx_v)
    gsems = (sem_g0, sem_g1)
    ssems = (sem_s0, sem_s1)

    def gather(j):
        return pltpu.make_async_copy(
            tbl_hbm.at[idx_v.at[pl.ds(j * CH, CH)]], rows_v.at[j % 2],
            gsems[j % 2])

    def drain(j):
        return pltpu.make_async_copy(
            rows_v.at[j % 2],
            atoms_hbm.at[pl.ds(row_base + j * CH, CH)],
            ssems[j % 2])

    for j in range(NCH):
        if j >= 2:
            drain(j - 2).wait()
        gather(j).start()
        gather(j).wait()
        drain(j).start()
    drain(NCH - 2).wait()
    drain(NCH - 1).wait()


_sc_gather = functools.partial(
    pl.kernel,
    mesh=plsc.VectorSubcoreMesh(core_axis_name="c", subcore_axis_name="s"),
    out_type=jax.ShapeDtypeStruct((N, ROW), jnp.float32),
    scratch_types=[
        pltpu.VMEM((TPW,), jnp.int32),
        pltpu.VMEM((2, CH, ROW), jnp.float32),
        pltpu.SemaphoreType.DMA,
        pltpu.SemaphoreType.DMA,
        pltpu.SemaphoreType.DMA,
        pltpu.SemaphoreType.DMA,
    ],
)(_sc_body)


def kernel(token_ids, digit_encoding, eos_embedding, empty_embedding,
           atom_offsets, log_weights_table):
    tok = token_ids.astype(jnp.int32)
    tbl3, lw_wide, tok_off = _prep(
        digit_encoding.astype(jnp.float32),
        eos_embedding.reshape(1, D).astype(jnp.float32),
        empty_embedding.reshape(1, D).astype(jnp.float32),
        atom_offsets.astype(jnp.float32),
        tok.reshape(LWROWS, TPR),
        log_weights_table.astype(jnp.float32),
        tok.reshape(NW, TPW),
    )
    atoms = _sc_gather(tbl3.reshape(NW * VPAD, ROW), tok_off)
    return atoms.reshape(B, S, A, D), lw_wide.reshape(B, S, A)


__all__ = ["kernel"]
